# baseline (speedup 1.0000x reference)
import functools

import jax
import jax.numpy as jnp
from jax import lax
from jax.experimental import pallas as pl
from jax.experimental.pallas import tpu as pltpu
from jax.experimental.pallas import tpu_sc as plsc

_TOPK = 30
_BITS = 30
_NC = 2
_NS = 16
_LANES = 16


def _fuse_kernel(beta_ref, a_ref, b_ref, c_ref, o_ref, *, blk):
    i = pl.program_id(0)
    x = (beta_ref[0] * a_ref[...] + beta_ref[1] * b_ref[...]
         + beta_ref[2] * c_ref[...])
    rows = jax.lax.broadcasted_iota(jnp.int32, x.shape, 0) + i * blk
    cols = jax.lax.broadcasted_iota(jnp.int32, x.shape, 1)
    o_ref[...] = jnp.where(rows == cols, 0.0, x)


def _sc_thr_body(fused_hbm, out_hbm, rowbuf, stage, thrbuf, sem,
                 *, n, rows_per, k, grp):
    nv = n // _LANES
    nch = nv // 4
    wid = lax.axis_index("s") * _NC + lax.axis_index("c")
    base = wid * rows_per
    lane = lax.iota(jnp.int32, _LANES)
    zero16i = jnp.zeros((_LANES,), jnp.int32)

    def process(slot, j, r):
        def pha(c, accs):
            a0, a1, a2, a3 = accs
            b = c * 4 * _LANES
            a0 = jnp.maximum(a0, rowbuf[slot, j, pl.ds(b, _LANES)])
            a1 = jnp.maximum(a1, rowbuf[slot, j, pl.ds(b + _LANES,
                                                       _LANES)])
            a2 = jnp.maximum(a2, rowbuf[slot, j, pl.ds(b + 2 * _LANES,
                                                       _LANES)])
            a3 = jnp.maximum(a3, rowbuf[slot, j, pl.ds(b + 3 * _LANES,
                                                       _LANES)])
            return a0, a1, a2, a3

        z = jnp.zeros((_LANES,), jnp.float32)
        a0, a1, a2, a3 = lax.fori_loop(0, nch, pha, (z, z, z, z))

        def coarse(it, lo):
            cand = lo | (1 << (_BITS - 1 - it))
            t = lax.bitcast_convert_type(cand, jnp.float32)
            cv = ((a0 >= t).astype(jnp.int32) + (a1 >= t).astype(jnp.int32)
                  + (a2 >= t).astype(jnp.int32)
                  + (a3 >= t).astype(jnp.int32))
            cnt = jnp.sum(cv)
            return jnp.where(cnt >= k, cand, lo)

        lo = lax.fori_loop(0, 12, coarse, zero16i)
        rmx = jnp.max(jnp.maximum(jnp.maximum(a0, a1),
                                  jnp.maximum(a2, a3)))
        hi = jnp.broadcast_to(
            lax.bitcast_convert_type(rmx, jnp.int32) + 1, (_LANES,))
        t0 = lax.bitcast_convert_type(lo, jnp.float32)

        def phb(c, cntv):
            for p in range(4):
                v = rowbuf[slot, j, pl.ds((c * 4 + p) * _LANES, _LANES)]
                msk = v > t0
                idx = (cntv << 4) + lane
                plsc.store_scatter(stage, [idx], v, mask=msk)
                cntv = cntv + msk.astype(jnp.int32)
            return cntv

        cntv = lax.fori_loop(0, nv // 4, phb, zero16i)
        mcnt = jnp.max(cntv)

        def fine_cond(carry):
            lo, hi = carry
            return jnp.max(hi - lo) > 1

        def fine(carry):
            lo, hi = carry
            mid = (lo + hi) >> 1
            t = lax.bitcast_convert_type(mid, jnp.float32)

            def cscan(jj, acc):
                v = stage[pl.ds(jj * _LANES, _LANES)]
                hit = jnp.logical_and(cntv > jj, v >= t)
                return acc + hit.astype(jnp.int32)

            cv = lax.fori_loop(0, mcnt, cscan, zero16i)
            cnt = jnp.sum(cv)
            ok = cnt >= k
            return (jnp.where(ok, mid, lo), jnp.where(ok, hi, mid))

        lo, hi = lax.while_loop(fine_cond, fine, (lo, hi))
        valv = lax.bitcast_convert_type(lo, jnp.float32)
        ridx = jnp.broadcast_to(jnp.int32(0) + r, (_LANES,))
        plsc.store_scatter(thrbuf, [ridx], valv, mask=lane == 0)

    ngrp = rows_per // grp

    def start(g, slot):
        pltpu.make_async_copy(
            fused_hbm.at[pl.ds(base + g * grp, grp)], rowbuf.at[slot],
            sem.at[slot]).start()

    def wait(g, slot):
        pltpu.make_async_copy(
            fused_hbm.at[pl.ds(base + g * grp, grp)], rowbuf.at[slot],
            sem.at[slot]).wait()

    def pgroup(slot, g):
        def prow(j, _):
            process(slot, j, g * grp + j)
            return 0

        lax.fori_loop(0, grp, prow, 0)

    start(0, 0)

    def outer(i, _):
        g0 = i * 2
        start(g0 + 1, 1)
        wait(g0, 0)
        pgroup(0, g0)

        @pl.when(i < ngrp // 2 - 1)
        def _():
            start(g0 + 2, 0)

        wait(g0 + 1, 1)
        pgroup(1, g0 + 1)
        return 0

    lax.fori_loop(0, ngrp // 2, outer, 0)
    pltpu.sync_copy(thrbuf, out_hbm.at[pl.ds(base, rows_per)])


def _sc_thresholds(fused, k):
    n = fused.shape[0]
    rows_per = n // (_NC * _NS)
    mesh = plsc.VectorSubcoreMesh(
        core_axis_name="c", subcore_axis_name="s",
        num_cores=_NC, num_subcores=_NS)
    kern = functools.partial(
        pl.kernel,
        out_type=jax.ShapeDtypeStruct((n,), jnp.float32),
        mesh=mesh,
        scratch_types=[
            pltpu.VMEM((2, 8, n), jnp.float32),
            pltpu.VMEM((n,), jnp.float32),
            pltpu.VMEM((rows_per,), jnp.float32),
            pltpu.SemaphoreType.DMA((2,)),
        ],
        compiler_params=pltpu.CompilerParams(needs_layout_passes=False),
    )(functools.partial(_sc_thr_body, n=n, rows_per=rows_per, k=k, grp=8))
    return kern(fused)


def _mask_sym_kernel(a_ref, ti_ref, b_ref, tj_ref, o_ref):
    x = a_ref[...]
    y = b_ref[...]
    sx = jnp.where(x >= ti_ref[...], x, 0.0)
    sy = jnp.where(y >= tj_ref[...], y, 0.0)
    o_ref[...] = jnp.maximum(sx, sy.T)


def _fuse_mask_kernel(beta_ref, a_ref, b_ref, c_ref, o_ref, x_ref,
                      *, k, blk):
    i = pl.program_id(0)
    x = (beta_ref[0] * a_ref[...] + beta_ref[1] * b_ref[...]
         + beta_ref[2] * c_ref[...])
    rows = jax.lax.broadcasted_iota(jnp.int32, x.shape, 0) + i * blk
    cols = jax.lax.broadcasted_iota(jnp.int32, x.shape, 1)
    x = jnp.where(rows == cols, 0.0, x)
    x_ref[...] = x

    nrows = x.shape[0]

    def body(it, lo):
        bit = _BITS - 1 - it
        cand = lo | (1 << bit)
        t = jax.lax.bitcast_convert_type(cand, jnp.float32)
        y = x_ref[...]
        cnt = jnp.sum((y >= t).astype(jnp.int32), axis=1, keepdims=True)
        return jnp.where(cnt >= k, cand, lo)

    lo = jax.lax.fori_loop(
        0, _BITS, body, jnp.zeros((nrows, 1), jnp.int32))
    t = jax.lax.bitcast_convert_type(lo, jnp.float32)
    y = x_ref[...]
    o_ref[...] = jnp.where(y >= t, y, 0.0).astype(jnp.bfloat16)


def _sym_kernel(a_ref, b_ref, o_ref):
    o_ref[...] = jnp.maximum(a_ref[...], b_ref[...].T).astype(jnp.float32)


def _sparsify(a, b, c, beta, k):
    n = a.shape[0]
    blk = 256
    s = pl.pallas_call(
        functools.partial(_fuse_mask_kernel, k=k, blk=blk),
        grid=(n // blk,),
        in_specs=[
            pl.BlockSpec(memory_space=pltpu.SMEM),
            pl.BlockSpec((blk, n), lambda i: (i, 0)),
            pl.BlockSpec((blk, n), lambda i: (i, 0)),
            pl.BlockSpec((blk, n), lambda i: (i, 0)),
        ],
        out_specs=pl.BlockSpec((blk, n), lambda i: (i, 0)),
        out_shape=jax.ShapeDtypeStruct((n, n), jnp.bfloat16),
        scratch_shapes=[pltpu.VMEM((blk, n), jnp.float32)],
    )(beta, a, b, c)
    t = 1024
    return pl.pallas_call(
        _sym_kernel,
        grid=(n // t, n // t),
        in_specs=[
            pl.BlockSpec((t, t), lambda i, j: (i, j)),
            pl.BlockSpec((t, t), lambda i, j: (j, i)),
        ],
        out_specs=pl.BlockSpec((t, t), lambda i, j: (i, j)),
        out_shape=jax.ShapeDtypeStruct((n, n), jnp.float32),
    )(s, s)


def _fuse_call(a, b, c, beta):
    n = a.shape[0]
    blk = 512
    return pl.pallas_call(
        functools.partial(_fuse_kernel, blk=blk),
        grid=(n // blk,),
        in_specs=[
            pl.BlockSpec(memory_space=pltpu.SMEM),
            pl.BlockSpec((blk, n), lambda i: (i, 0)),
            pl.BlockSpec((blk, n), lambda i: (i, 0)),
            pl.BlockSpec((blk, n), lambda i: (i, 0)),
        ],
        out_specs=pl.BlockSpec((blk, n), lambda i: (i, 0)),
        out_shape=jax.ShapeDtypeStruct((n, n), jnp.float32),
    )(beta, a, b, c)


def _mask_sym_call(fused, thr):
    n = fused.shape[0]
    t = 1024
    thr = thr.reshape(n, 1)
    return pl.pallas_call(
        _mask_sym_kernel,
        grid=(n // t, n // t),
        in_specs=[
            pl.BlockSpec((t, t), lambda i, j: (i, j)),
            pl.BlockSpec((t, 1), lambda i, j: (i, 0)),
            pl.BlockSpec((t, t), lambda i, j: (j, i)),
            pl.BlockSpec((t, 1), lambda i, j: (j, 0)),
        ],
        out_specs=pl.BlockSpec((t, t), lambda i, j: (i, j)),
        out_shape=jax.ShapeDtypeStruct((n, n), jnp.float32),
    )(fused, thr, fused, thr)


def kernel(mm_f, mm_s, mm_g, dd_t, dd_s, dd_g, att_m, att_d):
    beta_m = jax.nn.softmax(att_m.reshape(3))
    beta_d = jax.nn.softmax(att_d.reshape(3))
    fused_d = _fuse_call(dd_t, dd_s, dd_g, beta_d)
    thr_d = _sc_thresholds(fused_d, _TOPK)
    m_out = _sparsify(mm_f, mm_s, mm_g, beta_m, _TOPK)
    d_out = _mask_sym_call(fused_d, thr_d)
    return (m_out, d_out)


# speedup vs baseline: 1.0013x; 1.0013x over previous
import functools

import jax
import jax.numpy as jnp
from jax import lax
from jax.experimental import pallas as pl
from jax.experimental.pallas import tpu as pltpu
from jax.experimental.pallas import tpu_sc as plsc

_TOPK = 30
_BITS = 30
_NC = 2
_NS = 16
_LANES = 16


def _fuse_kernel(beta_ref, a_ref, b_ref, c_ref, o_ref, *, blk):
    i = pl.program_id(0)
    x = (beta_ref[0] * a_ref[...] + beta_ref[1] * b_ref[...]
         + beta_ref[2] * c_ref[...])
    rows = jax.lax.broadcasted_iota(jnp.int32, x.shape, 0) + i * blk
    cols = jax.lax.broadcasted_iota(jnp.int32, x.shape, 1)
    o_ref[...] = jnp.where(rows == cols, 0.0, x)


def _sc_thr_body(fused_hbm, out_hbm, rowbuf, stage, thrbuf, sem,
                 *, n, rows_per, k, grp):
    nv = n // _LANES
    nch = nv // 4
    wid = lax.axis_index("s") * _NC + lax.axis_index("c")
    base = wid * rows_per
    lane = lax.iota(jnp.int32, _LANES)
    zero16i = jnp.zeros((_LANES,), jnp.int32)

    def process(slot, j, r):
        def pha(c, accs):
            a0, a1, a2, a3 = accs
            b = c * 4 * _LANES
            a0 = jnp.maximum(a0, rowbuf[slot, j, pl.ds(b, _LANES)])
            a1 = jnp.maximum(a1, rowbuf[slot, j, pl.ds(b + _LANES,
                                                       _LANES)])
            a2 = jnp.maximum(a2, rowbuf[slot, j, pl.ds(b + 2 * _LANES,
                                                       _LANES)])
            a3 = jnp.maximum(a3, rowbuf[slot, j, pl.ds(b + 3 * _LANES,
                                                       _LANES)])
            return a0, a1, a2, a3

        z = jnp.zeros((_LANES,), jnp.float32)
        a0, a1, a2, a3 = lax.fori_loop(0, nch, pha, (z, z, z, z))

        def coarse(it, lo):
            cand = lo | (1 << (_BITS - 1 - it))
            t = lax.bitcast_convert_type(cand, jnp.float32)
            cv = ((a0 >= t).astype(jnp.int32) + (a1 >= t).astype(jnp.int32)
                  + (a2 >= t).astype(jnp.int32)
                  + (a3 >= t).astype(jnp.int32))
            cnt = jnp.sum(cv)
            return jnp.where(cnt >= k, cand, lo)

        lo = lax.fori_loop(0, 12, coarse, zero16i)
        rmx = jnp.max(jnp.maximum(jnp.maximum(a0, a1),
                                  jnp.maximum(a2, a3)))
        hi = jnp.broadcast_to(
            lax.bitcast_convert_type(rmx, jnp.int32) + 1, (_LANES,))
        t0 = lax.bitcast_convert_type(lo, jnp.float32)

        def phb(c, cntv):
            for p in range(4):
                v = rowbuf[slot, j, pl.ds((c * 4 + p) * _LANES, _LANES)]
                msk = v > t0
                idx = (cntv << 4) + lane
                plsc.store_scatter(stage, [idx], v, mask=msk)
                cntv = cntv + msk.astype(jnp.int32)
            return cntv

        cntv = lax.fori_loop(0, nv // 4, phb, zero16i)
        mcnt = jnp.max(cntv)

        def fine_cond(carry):
            lo, hi = carry
            return jnp.max(hi - lo) > 1

        def fine(carry):
            lo, hi = carry
            mid = (lo + hi) >> 1
            t = lax.bitcast_convert_type(mid, jnp.float32)

            def cscan(jj, acc):
                v = stage[pl.ds(jj * _LANES, _LANES)]
                hit = jnp.logical_and(cntv > jj, v >= t)
                return acc + hit.astype(jnp.int32)

            cv = lax.fori_loop(0, mcnt, cscan, zero16i)
            cnt = jnp.sum(cv)
            ok = cnt >= k
            return (jnp.where(ok, mid, lo), jnp.where(ok, hi, mid))

        lo, hi = lax.while_loop(fine_cond, fine, (lo, hi))
        valv = lax.bitcast_convert_type(lo, jnp.float32)
        ridx = jnp.broadcast_to(jnp.int32(0) + r, (_LANES,))
        plsc.store_scatter(thrbuf, [ridx], valv, mask=lane == 0)

    ngrp = rows_per // grp

    def start(g, slot):
        pltpu.make_async_copy(
            fused_hbm.at[pl.ds(base + g * grp, grp)], rowbuf.at[slot],
            sem.at[slot]).start()

    def wait(g, slot):
        pltpu.make_async_copy(
            fused_hbm.at[pl.ds(base + g * grp, grp)], rowbuf.at[slot],
            sem.at[slot]).wait()

    def pgroup(slot, g):
        def prow(j, _):
            process(slot, j, g * grp + j)
            return 0

        lax.fori_loop(0, grp, prow, 0)

    start(0, 0)

    def outer(i, _):
        g0 = i * 2
        start(g0 + 1, 1)
        wait(g0, 0)
        pgroup(0, g0)

        @pl.when(i < ngrp // 2 - 1)
        def _():
            start(g0 + 2, 0)

        wait(g0 + 1, 1)
        pgroup(1, g0 + 1)
        return 0

    lax.fori_loop(0, ngrp // 2, outer, 0)
    pltpu.sync_copy(thrbuf, out_hbm.at[pl.ds(base, rows_per)])


def _sc_thresholds(fused, k):
    n = fused.shape[0]
    rows_per = n // (_NC * _NS)
    mesh = plsc.VectorSubcoreMesh(
        core_axis_name="c", subcore_axis_name="s",
        num_cores=_NC, num_subcores=_NS)
    kern = functools.partial(
        pl.kernel,
        out_type=jax.ShapeDtypeStruct((n,), jnp.float32),
        mesh=mesh,
        scratch_types=[
            pltpu.VMEM((2, 8, n), jnp.float32),
            pltpu.VMEM((n,), jnp.float32),
            pltpu.VMEM((rows_per,), jnp.float32),
            pltpu.SemaphoreType.DMA((2,)),
        ],
        compiler_params=pltpu.CompilerParams(needs_layout_passes=False),
    )(functools.partial(_sc_thr_body, n=n, rows_per=rows_per, k=k, grp=8))
    return kern(fused)


def _mask_sym_kernel(a_ref, ti_ref, b_ref, tj_ref, o_ref):
    x = a_ref[...]
    y = b_ref[...]
    sx = jnp.where(x >= ti_ref[...], x, 0.0)
    sy = jnp.where(y >= tj_ref[...], y, 0.0)
    o_ref[...] = jnp.maximum(sx, sy.T)


def _fuse_mask_kernel(beta_ref, a_ref, b_ref, c_ref, o_ref, x_ref,
                      *, k, blk):
    i = pl.program_id(0)
    x = (beta_ref[0] * a_ref[...] + beta_ref[1] * b_ref[...]
         + beta_ref[2] * c_ref[...])
    rows = jax.lax.broadcasted_iota(jnp.int32, x.shape, 0) + i * blk
    cols = jax.lax.broadcasted_iota(jnp.int32, x.shape, 1)
    x = jnp.where(rows == cols, 0.0, x)
    x_ref[...] = x

    nrows = x.shape[0]

    def body(it, lo):
        bit = _BITS - 1 - it
        cand = lo | (1 << bit)
        t = jax.lax.bitcast_convert_type(cand, jnp.float32)
        y = x_ref[...]
        cnt = jnp.sum((y >= t).astype(jnp.int32), axis=1, keepdims=True)
        return jnp.where(cnt >= k, cand, lo)

    lo = jax.lax.fori_loop(
        0, _BITS, body, jnp.zeros((nrows, 1), jnp.int32))
    t = jax.lax.bitcast_convert_type(lo, jnp.float32)
    y = x_ref[...]
    o_ref[...] = jnp.where(y >= t, y, 0.0).astype(jnp.bfloat16)


def _sym_kernel(a_ref, b_ref, o_ref):
    o_ref[...] = jnp.maximum(a_ref[...], b_ref[...].T).astype(jnp.float32)


def _sparsify(a, b, c, beta, k):
    n = a.shape[0]
    blk = 256
    s = pl.pallas_call(
        functools.partial(_fuse_mask_kernel, k=k, blk=blk),
        grid=(n // blk,),
        in_specs=[
            pl.BlockSpec(memory_space=pltpu.SMEM),
            pl.BlockSpec((blk, n), lambda i: (i, 0)),
            pl.BlockSpec((blk, n), lambda i: (i, 0)),
            pl.BlockSpec((blk, n), lambda i: (i, 0)),
        ],
        out_specs=pl.BlockSpec((blk, n), lambda i: (i, 0)),
        out_shape=jax.ShapeDtypeStruct((n, n), jnp.bfloat16),
        scratch_shapes=[pltpu.VMEM((blk, n), jnp.float32)],
    )(beta, a, b, c)
    t = 1024
    return pl.pallas_call(
        _sym_kernel,
        grid=(n // t, n // t),
        in_specs=[
            pl.BlockSpec((t, t), lambda i, j: (i, j)),
            pl.BlockSpec((t, t), lambda i, j: (j, i)),
        ],
        out_specs=pl.BlockSpec((t, t), lambda i, j: (i, j)),
        out_shape=jax.ShapeDtypeStruct((n, n), jnp.float32),
    )(s, s)


def _fuse_call(a, b, c, beta):
    n = a.shape[0]
    blk = 256
    return pl.pallas_call(
        functools.partial(_fuse_kernel, blk=blk),
        grid=(n // blk,),
        in_specs=[
            pl.BlockSpec(memory_space=pltpu.SMEM),
            pl.BlockSpec((blk, n), lambda i: (i, 0)),
            pl.BlockSpec((blk, n), lambda i: (i, 0)),
            pl.BlockSpec((blk, n), lambda i: (i, 0)),
        ],
        out_specs=pl.BlockSpec((blk, n), lambda i: (i, 0)),
        out_shape=jax.ShapeDtypeStruct((n, n), jnp.float32),
    )(beta, a, b, c)


def _mask_sym_call(fused, thr):
    n = fused.shape[0]
    t = 1024
    thr = thr.reshape(n, 1)
    return pl.pallas_call(
        _mask_sym_kernel,
        grid=(n // t, n // t),
        in_specs=[
            pl.BlockSpec((t, t), lambda i, j: (i, j)),
            pl.BlockSpec((t, 1), lambda i, j: (i, 0)),
            pl.BlockSpec((t, t), lambda i, j: (j, i)),
            pl.BlockSpec((t, 1), lambda i, j: (j, 0)),
        ],
        out_specs=pl.BlockSpec((t, t), lambda i, j: (i, j)),
        out_shape=jax.ShapeDtypeStruct((n, n), jnp.float32),
    )(fused, thr, fused, thr)


def kernel(mm_f, mm_s, mm_g, dd_t, dd_s, dd_g, att_m, att_d):
    beta_m = jax.nn.softmax(att_m.reshape(3))
    beta_d = jax.nn.softmax(att_d.reshape(3))
    fused_d = _fuse_call(dd_t, dd_s, dd_g, beta_d)
    thr_d = _sc_thresholds(fused_d, _TOPK)
    m_out = _sparsify(mm_f, mm_s, mm_g, beta_m, _TOPK)
    d_out = _mask_sym_call(fused_d, thr_d)
    return (m_out, d_out)
